# Initial kernel scaffold; baseline (speedup 1.0000x reference)
import jax
import jax.numpy as jnp
from jax.experimental import pallas as pl


def kernel(x, table):
    raise NotImplementedError("write your pallas kernel here")


# speedup vs baseline: 1.5072x; 1.5072x over previous
import functools

import jax
import jax.numpy as jnp
from jax import lax
from jax.experimental import pallas as pl
from jax.experimental.pallas import tpu as pltpu
from jax.experimental.pallas import tpu_sc as plsc

_EMBED_DIM = 32

_info = plsc.get_sparse_core_info()
_NC, _NS = _info.num_cores, _info.num_subcores
_NW = _NC * _NS


def _make_gather(n_rows: int, chunk: int):
    assert n_rows % (_NW * chunk) == 0
    n_chunks = n_rows // (_NW * chunk)
    b_per_w = n_rows // _NW
    mesh = plsc.VectorSubcoreMesh(core_axis_name="c", subcore_axis_name="s")

    @functools.partial(
        pl.kernel,
        mesh=mesh,
        compiler_params=pltpu.CompilerParams(use_tc_tiling_on_sc=False),
        out_type=jax.ShapeDtypeStruct((n_rows, _EMBED_DIM), jnp.float32),
        scratch_types=[
            pltpu.VMEM((chunk,), jnp.int32),
            pltpu.VMEM((chunk, _EMBED_DIM), jnp.float32),
            pltpu.SemaphoreType.DMA,
        ],
    )
    def gather_kernel(table_hbm, idx_hbm, out_hbm, idx_v, rows_v, sem):
        wid = lax.axis_index("s") * _NC + lax.axis_index("c")
        base = wid * b_per_w

        def body(i, carry):
            off = base + i * chunk
            pltpu.sync_copy(idx_hbm.at[pl.ds(off, chunk)], idx_v)
            pltpu.async_copy(table_hbm.at[idx_v], rows_v, sem).wait()
            pltpu.sync_copy(rows_v, out_hbm.at[pl.ds(off, chunk)])
            return carry

        lax.fori_loop(0, n_chunks, body, 0)

    return gather_kernel


def kernel(x, table):
    batch, hist = x.shape
    n_rows = batch * hist
    idx = x.reshape(n_rows).astype(jnp.int32)
    out = _make_gather(n_rows, 2048)(table, idx)
    return out.reshape(batch, hist, _EMBED_DIM)


# speedup vs baseline: 1.5139x; 1.0045x over previous
import functools

import jax
import jax.numpy as jnp
from jax import lax
from jax.experimental import pallas as pl
from jax.experimental.pallas import tpu as pltpu
from jax.experimental.pallas import tpu_sc as plsc

_EMBED_DIM = 32

_info = plsc.get_sparse_core_info()
_NC, _NS = _info.num_cores, _info.num_subcores
_NW = _NC * _NS


def _make_gather(n_rows: int, chunk: int):
    assert n_rows % (_NW * chunk) == 0
    n_chunks = n_rows // (_NW * chunk)
    b_per_w = n_rows // _NW
    mesh = plsc.VectorSubcoreMesh(core_axis_name="c", subcore_axis_name="s")

    @functools.partial(
        pl.kernel,
        mesh=mesh,
        compiler_params=pltpu.CompilerParams(use_tc_tiling_on_sc=False),
        out_type=jax.ShapeDtypeStruct((n_rows, _EMBED_DIM), jnp.float32),
        scratch_types=[
            pltpu.VMEM((n_chunks, chunk), jnp.int32),
            pltpu.VMEM((2, chunk, _EMBED_DIM), jnp.float32),
            pltpu.SemaphoreType.DMA,
            pltpu.SemaphoreType.DMA,
            pltpu.SemaphoreType.DMA,
            pltpu.SemaphoreType.DMA,
        ],
    )
    def gather_kernel(table_hbm, idx_hbm, out_hbm, idx_v, rows_v, g0, g1, s0, s1):
        wid = lax.axis_index("s") * _NC + lax.axis_index("c")
        base = wid * b_per_w
        pltpu.sync_copy(idx_hbm.at[wid], idx_v)
        gsem = (g0, g1)
        ssem = (s0, s1)
        gathers = [None, None]
        stores = [None, None]
        gathers[0] = pltpu.async_copy(table_hbm.at[idx_v.at[0]], rows_v.at[0], g0)
        for i in range(n_chunks):
            b = i % 2
            nb = (i + 1) % 2
            if i + 1 < n_chunks:
                if stores[nb] is not None:
                    stores[nb].wait()
                gathers[nb] = pltpu.async_copy(
                    table_hbm.at[idx_v.at[i + 1]], rows_v.at[nb], gsem[nb]
                )
            gathers[b].wait()
            stores[b] = pltpu.async_copy(
                rows_v.at[b], out_hbm.at[pl.ds(base + i * chunk, chunk)], ssem[b]
            )
        stores[(n_chunks - 1) % 2].wait()
        if n_chunks >= 2:
            stores[(n_chunks - 2) % 2].wait()

    return gather_kernel


def kernel(x, table):
    batch, hist = x.shape
    n_rows = batch * hist
    chunk = 1280
    n_chunks = n_rows // (_NW * chunk)
    idx = x.reshape(_NW, n_chunks, chunk).astype(jnp.int32)
    out = _make_gather(n_rows, chunk)(table, idx)
    return out.reshape(batch, hist, _EMBED_DIM)
